# baseline (speedup 1.0000x reference)
import jax
import jax.numpy as jnp
from jax import lax
from jax.experimental import pallas as pl
from jax.experimental.pallas import tpu as pltpu
from jax.experimental.pallas import tpu_sc as plsc

B, V, D, P = 64, 7, 1024, 64
IN = D * P
OUT = 96
E, K, R = 16, 2, 8
H = D // 2
SCALING = 16 / R

N = B * V
DC = 128
NSTEPS = D // DC
CHUNK = DC * P

_NT = (((1,), (1,)), ((), ()))


def _stage1_body(x_ref, wb_ref, a_ref, base_ref, temp_ref, pool_ref):
    i = pl.program_id(0)
    x4 = x_ref[...]
    x16 = x4.astype(jnp.bfloat16).swapaxes(2, 3)
    xb16 = x16.reshape(N, CHUNK)
    b_part = jax.lax.dot_general(xb16, wb_ref[...].astype(jnp.bfloat16), _NT,
                                 preferred_element_type=jnp.float32)
    t_part = jax.lax.dot_general(xb16, a_ref[...].astype(jnp.bfloat16), _NT,
                                 preferred_element_type=jnp.float32)
    ps = x4.sum(axis=2).sum(axis=1)
    pool_ref[0] = ps * (1.0 / (V * P))

    @pl.when(i == 0)
    def _init():
        base_ref[...] = b_part
        temp_ref[...] = t_part

    @pl.when(i != 0)
    def _acc():
        base_ref[...] += b_part
        temp_ref[...] += t_part


def _mlp_body(pool_ref, w1_ref, b1_ref, w2_ref, b2_ref, logits_ref):
    pooled = pool_ref[...]
    h = jax.lax.dot_general(pooled, w1_ref[...], _NT,
                            preferred_element_type=jnp.float32)
    h = jnp.maximum(h + b1_ref[...], 0.0)
    logits_ref[...] = jax.lax.dot_general(
        h, w2_ref[...], _NT,
        preferred_element_type=jnp.float32) + b2_ref[...]


def _route_rows(logits_hbm, probs_hbm, we_hbm, lvm, pvm, wvm):
    info = plsc.get_sparse_core_info()
    nc = info.num_cores
    wid = lax.axis_index("s") * nc + lax.axis_index("c")
    rows = B // 8

    @pl.when(wid < 8)
    def _work():
        pltpu.sync_copy(logits_hbm.at[pl.ds(wid * rows, rows)], lvm)
        eidx = lax.iota(jnp.int32, E)
        _dn = jax.lax.GatherDimensionNumbers(
            offset_dims=(), collapsed_slice_dims=(0,), start_index_map=(0,))

        def lanes(u, k):
            return jax.lax.gather(
                u, (eidx ^ k)[:, None], _dn, (1,),
                mode=jax.lax.GatherScatterMode.PROMISE_IN_BOUNDS)

        def allred(u, op):
            for k in (1, 2, 4, 8):
                u = op(u, lanes(u, k))
            return u

        for r in range(rows):
            v = lvm[r]
            ex = jnp.exp(v - allred(v, jnp.maximum))
            probs = ex / allred(ex, jnp.add)
            m1 = allred(probs, jnp.maximum)
            i1 = allred(jnp.where(probs == m1, eidx, E), jnp.minimum)
            masked = jnp.where(eidx == i1, -1.0, probs)
            m2 = allred(masked, jnp.maximum)
            i2 = allred(jnp.where(masked == m2, eidx, E), jnp.minimum)
            s = jnp.maximum(m1 + m2, 1e-6)
            w = jnp.where(eidx == i1, m1 / s, 0.0) + \
                jnp.where(eidx == i2, m2 / s, 0.0)
            pvm[r] = probs
            wvm[r] = w
        pltpu.sync_copy(pvm, probs_hbm.at[pl.ds(wid * rows, rows)])
        pltpu.sync_copy(wvm, we_hbm.at[pl.ds(wid * rows, rows)])


def _delta_body(temp_ref, lb_ref, delta_ref):
    temp = temp_ref[...]
    for e in range(E):
        te = temp[:, e * R:(e + 1) * R]
        delta_ref[:, e * OUT:(e + 1) * OUT] = jax.lax.dot_general(
            te, lb_ref[e], _NT, preferred_element_type=jnp.float32)


def _stage2_body(base_ref, delta_ref, we_ref, bb_ref, out_ref):
    rn = jax.lax.broadcasted_iota(jnp.int32, (N, B), 0) // V
    cb = jax.lax.broadcasted_iota(jnp.int32, (N, B), 1)
    sel = (rn == cb).astype(jnp.float32)
    w_rows = jnp.dot(sel, we_ref[...], preferred_element_type=jnp.float32)

    moe = jnp.zeros((N, OUT), dtype=jnp.float32)
    for e in range(E):
        moe += w_rows[:, e:e + 1] * delta_ref[:, e * OUT:(e + 1) * OUT]
    out_ref[...] = base_ref[...] + bb_ref[...] + moe * SCALING


@jax.jit
def _run(x, W_base, b_base, W1, b1, W2, b2, lora_A, lora_B):
    xt = jnp.transpose(x, (0, 1, 3, 2))
    A2 = lora_A.reshape(E * R, IN)

    base_acc, temp_acc, pooled = pl.pallas_call(
        _stage1_body,
        grid=(NSTEPS,),
        in_specs=[
            pl.BlockSpec((B, V, P, DC), lambda i: (0, 0, 0, i)),
            pl.BlockSpec((OUT, CHUNK), lambda i: (0, i)),
            pl.BlockSpec((E * R, CHUNK), lambda i: (0, i)),
        ],
        out_specs=[
            pl.BlockSpec((N, OUT), lambda i: (0, 0)),
            pl.BlockSpec((N, E * R), lambda i: (0, 0)),
            pl.BlockSpec((1, B, DC), lambda i: (i, 0, 0)),
        ],
        out_shape=[
            jax.ShapeDtypeStruct((N, OUT), jnp.float32),
            jax.ShapeDtypeStruct((N, E * R), jnp.float32),
            jax.ShapeDtypeStruct((NSTEPS, B, DC), jnp.float32),
        ],
    )(xt, W_base, A2)
    pooled = pooled.transpose(1, 0, 2).reshape(B, D)

    logits = pl.pallas_call(
        _mlp_body,
        out_shape=jax.ShapeDtypeStruct((B, E), jnp.float32),
    )(pooled, W1, b1.reshape(1, H), W2, b2.reshape(1, E))

    mesh = plsc.VectorSubcoreMesh(core_axis_name="c", subcore_axis_name="s")
    probs, w_e = pl.kernel(
        _route_rows,
        mesh=mesh,
        out_type=[
            jax.ShapeDtypeStruct((B, E), jnp.float32),
            jax.ShapeDtypeStruct((B, E), jnp.float32),
        ],
        scratch_types=[
            pltpu.VMEM((B // 8, E), jnp.float32),
            pltpu.VMEM((B // 8, E), jnp.float32),
            pltpu.VMEM((B // 8, E), jnp.float32),
        ],
    )(logits)

    delta_all = pl.pallas_call(
        _delta_body,
        out_shape=jax.ShapeDtypeStruct((N, E * OUT), jnp.float32),
    )(temp_acc, lora_B)

    final = pl.pallas_call(
        _stage2_body,
        out_shape=jax.ShapeDtypeStruct((N, OUT), jnp.float32),
    )(base_acc, delta_all, w_e, b_base.reshape(1, OUT))
    return final.reshape(B, V, OUT), probs


def kernel(x, W_base, b_base, W1, b1, W2, b2, lora_A, lora_B):
    return _run(x, W_base, b_base, W1, b1, W2, b2, lora_A, lora_B)


# speedup vs baseline: 1.0374x; 1.0374x over previous
import jax
import jax.numpy as jnp
from jax import lax
from jax.experimental import pallas as pl
from jax.experimental.pallas import tpu as pltpu
from jax.experimental.pallas import tpu_sc as plsc

B, V, D, P = 64, 7, 1024, 64
IN = D * P
OUT = 96
E, K, R = 16, 2, 8
H = D // 2
SCALING = 16 / R

N = B * V
DC = 128
NSTEPS = D // DC
CHUNK = DC * P

_NT = (((1,), (1,)), ((), ()))


def _stage1_body(x_ref, wb_ref, a_ref, base_ref, temp_ref, pool_ref):
    i = pl.program_id(0)
    x4 = x_ref[...]
    x16 = x4.astype(jnp.bfloat16).swapaxes(2, 3)
    xb16 = x16.reshape(N, CHUNK)
    b_part = jax.lax.dot_general(xb16, wb_ref[...].astype(jnp.bfloat16), _NT,
                                 preferred_element_type=jnp.float32)
    t_part = jax.lax.dot_general(xb16, a_ref[...].astype(jnp.bfloat16), _NT,
                                 preferred_element_type=jnp.float32)
    ps = x4.sum(axis=2).sum(axis=1)
    pool_ref[0] = ps * (1.0 / (V * P))

    @pl.when(i == 0)
    def _init():
        base_ref[...] = b_part
        temp_ref[...] = t_part

    @pl.when(i != 0)
    def _acc():
        base_ref[...] += b_part
        temp_ref[...] += t_part


def _mlp_body(pool_ref, w1_ref, b1_ref, w2_ref, b2_ref, logits_ref):
    pooled = pool_ref[...]
    h = jax.lax.dot_general(pooled, w1_ref[...], _NT,
                            preferred_element_type=jnp.float32)
    h = jnp.maximum(h + b1_ref[...], 0.0)
    logits_ref[...] = jax.lax.dot_general(
        h, w2_ref[...], _NT,
        preferred_element_type=jnp.float32) + b2_ref[...]


def _route_rows(logits_hbm, probs_hbm, we_hbm, lvm, pvm, wvm):
    info = plsc.get_sparse_core_info()
    nc = info.num_cores
    wid = lax.axis_index("s") * nc + lax.axis_index("c")
    rows = B // 8

    @pl.when(wid < 8)
    def _work():
        pltpu.sync_copy(logits_hbm.at[pl.ds(wid * rows, rows)], lvm)
        eidx = lax.iota(jnp.int32, E)
        _dn = jax.lax.GatherDimensionNumbers(
            offset_dims=(), collapsed_slice_dims=(0,), start_index_map=(0,))

        def lanes(u, k):
            return jax.lax.gather(
                u, (eidx ^ k)[:, None], _dn, (1,),
                mode=jax.lax.GatherScatterMode.PROMISE_IN_BOUNDS)

        def allred(u, op):
            for k in (1, 2, 4, 8):
                u = op(u, lanes(u, k))
            return u

        for r in range(rows):
            v = lvm[r]
            ex = jnp.exp(v - allred(v, jnp.maximum))
            probs = ex / allred(ex, jnp.add)
            m1 = allred(probs, jnp.maximum)
            i1 = allred(jnp.where(probs == m1, eidx, E), jnp.minimum)
            masked = jnp.where(eidx == i1, -1.0, probs)
            m2 = allred(masked, jnp.maximum)
            i2 = allred(jnp.where(masked == m2, eidx, E), jnp.minimum)
            s = jnp.maximum(m1 + m2, 1e-6)
            w = jnp.where(eidx == i1, m1 / s, 0.0) + \
                jnp.where(eidx == i2, m2 / s, 0.0)
            pvm[r] = probs
            wvm[r] = w
        pltpu.sync_copy(pvm, probs_hbm.at[pl.ds(wid * rows, rows)])
        pltpu.sync_copy(wvm, we_hbm.at[pl.ds(wid * rows, rows)])


def _stage2_body(base_ref, temp_ref, we_ref, bb_ref, lb_ref, out_ref):
    rn = jax.lax.broadcasted_iota(jnp.int32, (N, B), 0) // V
    cb = jax.lax.broadcasted_iota(jnp.int32, (N, B), 1)
    sel = (rn == cb).astype(jnp.float32)
    w_rows = jnp.dot(sel, we_ref[...], preferred_element_type=jnp.float32)

    temp = temp_ref[...]
    moe = jnp.zeros((N, OUT), dtype=jnp.float32)
    for e in range(E):
        te = temp[:, e * R:(e + 1) * R]
        de = jax.lax.dot_general(te, lb_ref[e], _NT,
                                 preferred_element_type=jnp.float32)
        moe += w_rows[:, e:e + 1] * de
    out_ref[...] = base_ref[...] + bb_ref[...] + moe * SCALING


@jax.jit
def _run(x, W_base, b_base, W1, b1, W2, b2, lora_A, lora_B):
    xt = jnp.transpose(x, (0, 1, 3, 2))
    A2 = lora_A.reshape(E * R, IN)

    base_acc, temp_acc, pooled = pl.pallas_call(
        _stage1_body,
        grid=(NSTEPS,),
        in_specs=[
            pl.BlockSpec((B, V, P, DC), lambda i: (0, 0, 0, i)),
            pl.BlockSpec((OUT, CHUNK), lambda i: (0, i)),
            pl.BlockSpec((E * R, CHUNK), lambda i: (0, i)),
        ],
        out_specs=[
            pl.BlockSpec((N, OUT), lambda i: (0, 0)),
            pl.BlockSpec((N, E * R), lambda i: (0, 0)),
            pl.BlockSpec((1, B, DC), lambda i: (i, 0, 0)),
        ],
        out_shape=[
            jax.ShapeDtypeStruct((N, OUT), jnp.float32),
            jax.ShapeDtypeStruct((N, E * R), jnp.float32),
            jax.ShapeDtypeStruct((NSTEPS, B, DC), jnp.float32),
        ],
    )(xt, W_base, A2)
    pooled = pooled.transpose(1, 0, 2).reshape(B, D)

    logits = pl.pallas_call(
        _mlp_body,
        out_shape=jax.ShapeDtypeStruct((B, E), jnp.float32),
    )(pooled, W1, b1.reshape(1, H), W2, b2.reshape(1, E))

    mesh = plsc.VectorSubcoreMesh(core_axis_name="c", subcore_axis_name="s")
    probs, w_e = pl.kernel(
        _route_rows,
        mesh=mesh,
        out_type=[
            jax.ShapeDtypeStruct((B, E), jnp.float32),
            jax.ShapeDtypeStruct((B, E), jnp.float32),
        ],
        scratch_types=[
            pltpu.VMEM((B // 8, E), jnp.float32),
            pltpu.VMEM((B // 8, E), jnp.float32),
            pltpu.VMEM((B // 8, E), jnp.float32),
        ],
    )(logits)

    final = pl.pallas_call(
        _stage2_body,
        out_shape=jax.ShapeDtypeStruct((N, OUT), jnp.float32),
    )(base_acc, temp_acc, w_e, b_base.reshape(1, OUT), lora_B)
    return final.reshape(B, V, OUT), probs


def kernel(x, W_base, b_base, W1, b1, W2, b2, lora_A, lora_B):
    return _run(x, W_base, b_base, W1, b1, W2, b2, lora_A, lora_B)
